# baseline (speedup 1.0000x reference)
import functools

import jax
import jax.numpy as jnp
from jax import lax
from jax.experimental import pallas as pl
from jax.experimental.pallas import tpu as pltpu
from jax.experimental.pallas import tpu_sc as plsc

N_NODES = 10000
N_EDGES = 320000
D = 128
ED = 16
H = 256

NC = 2
NS = 16
NW = NC * NS
SPLIT = 2

GCH = 128


def _sc_gather_body(table_hbm, idx_hbm, out_hbm,
                    idx_a, idx_b, rows_a, rows_b, sem_a, sem_b):
    c = lax.axis_index("c")
    s = lax.axis_index("s")
    wid = s * NC + c
    n_total = idx_hbm.shape[0]
    per_w = n_total // NW
    base = wid * per_w
    n_pairs = per_w // GCH // 2

    def chunk(j):
        return pl.ds(pl.multiple_of(base + j * GCH, GCH), GCH)

    pltpu.sync_copy(idx_hbm.at[chunk(0)], idx_a)
    pltpu.async_copy(table_hbm.at[idx_a], rows_a, sem_a)

    @pl.loop(0, n_pairs)
    def _(jj):
        j = jj * 2
        pltpu.sync_copy(idx_hbm.at[chunk(j + 1)], idx_b)
        pltpu.async_copy(table_hbm.at[idx_b], rows_b, sem_b)
        pltpu.make_async_copy(table_hbm.at[idx_a], rows_a, sem_a).wait()
        pltpu.sync_copy(rows_a, out_hbm.at[chunk(j)])

        @pl.when(jj < n_pairs - 1)
        def _():
            pltpu.sync_copy(idx_hbm.at[chunk(j + 2)], idx_a)
            pltpu.async_copy(table_hbm.at[idx_a], rows_a, sem_a)

        pltpu.make_async_copy(table_hbm.at[idx_b], rows_b, sem_b).wait()
        pltpu.sync_copy(rows_b, out_hbm.at[chunk(j + 1)])


TW = 64


def _sc_gather(table, idx):
    b = idx.shape[0]
    mesh = plsc.VectorSubcoreMesh(core_axis_name="c", subcore_axis_name="s")
    return pl.kernel(
        _sc_gather_body,
        out_type=jax.ShapeDtypeStruct((b, TW), jnp.int32),
        mesh=mesh,
        compiler_params=pltpu.CompilerParams(use_tc_tiling_on_sc=False),
        scratch_types=[
            pltpu.VMEM((GCH,), jnp.int32),
            pltpu.VMEM((GCH,), jnp.int32),
            pltpu.VMEM((GCH, TW), jnp.int32),
            pltpu.VMEM((GCH, TW), jnp.int32),
            pltpu.SemaphoreType.DMA,
            pltpu.SemaphoreType.DMA,
        ],
    )(table, idx)


SCH = 40
N_PAD = 10240
STRIPE = N_PAD // NS


def _sc_scatter_body(n_edges, msg_hbm, dst_hbm, zeros_hbm, out_hbm,
                     idx_v, rows_v, acc_sh):
    c = lax.axis_index("c")
    s = lax.axis_index("s")
    per_core = n_edges // NC
    per_tile = per_core // NS
    base = c * per_core + s * per_tile
    n_sub = STRIPE // SCH

    pltpu.sync_copy(zeros_hbm, rows_v)

    @pl.loop(0, n_sub)
    def _(k):
        off = pl.multiple_of(s * STRIPE + k * SCH, 8)
        pltpu.sync_copy(rows_v, acc_sh.at[pl.ds(off, SCH)])

    plsc.subcore_barrier()

    @pl.loop(0, per_tile // SCH)
    def _(j):
        off = pl.multiple_of(base + j * SCH, 8)
        pltpu.sync_copy(dst_hbm.at[pl.ds(off, SCH)], idx_v)
        pltpu.sync_copy(msg_hbm.at[pl.ds(off, SCH)], rows_v)
        pltpu.sync_copy(rows_v, acc_sh.at[idx_v], add=True)

    plsc.subcore_barrier()

    @pl.loop(0, n_sub)
    def _(k):
        off = pl.multiple_of(s * STRIPE + k * SCH, 8)
        pltpu.sync_copy(acc_sh.at[pl.ds(off, SCH)], rows_v)
        pltpu.sync_copy(rows_v, out_hbm.at[c, pl.ds(off, SCH)])


def _sc_segment_sum(msg, dst_idx, zeros_stripe):
    mesh = plsc.VectorSubcoreMesh(core_axis_name="c", subcore_axis_name="s")
    return pl.kernel(
        functools.partial(_sc_scatter_body, msg.shape[0]),
        out_type=jax.ShapeDtypeStruct((NC, N_PAD, D), jnp.float32),
        mesh=mesh,
        scratch_types=[
            pltpu.VMEM((SCH,), jnp.int32),
            pltpu.VMEM((SCH, D), jnp.float32),
            pltpu.VMEM_SHARED((N_PAD, D), jnp.float32),
        ],
    )(msg, dst_idx, zeros_stripe)


BLK = 1280


def _silu(x):
    return x * jax.nn.sigmoid(x)


def _unpack_pairs(x_i32):
    f32 = jnp.float32
    bf = jnp.bfloat16
    even = lax.bitcast_convert_type(jnp.left_shift(x_i32, 16), f32).astype(bf)
    odd = lax.bitcast_convert_type(
        jnp.bitwise_and(x_i32, jnp.int32(-65536)), f32).astype(bf)
    return jnp.concatenate([even, odd], axis=1)


def _sigmoid_t(x):
    return 0.5 * jnp.tanh(x * 0.5) + 0.5


def _silu_t(x):
    return x * _sigmoid_t(x)


def _mlp_body(src, dst, ef, ew,
              w0sd, w0e, b0, gw1, gb1, ow1, ob1, gw2, gb2, ow2, ob2,
              msg_out):
    f32 = jnp.float32
    bf = jnp.bfloat16
    x = jnp.concatenate(
        [_unpack_pairs(src[...]), _unpack_pairs(dst[...])], axis=1)
    pre0 = (jnp.dot(x, w0sd[...], preferred_element_type=f32)
            + jnp.dot(ef[...], w0e[...], preferred_element_type=f32)
            + b0[...])
    a1 = _silu_t(pre0.astype(bf))
    g1 = _silu_t((jnp.dot(a1[:, :H], gw1[...], preferred_element_type=f32) + gb1[...]).astype(bf))
    o1 = _silu_t((jnp.dot(a1[:, H:], ow1[...], preferred_element_type=f32) + ob1[...]).astype(bf))
    gp = (jnp.dot(g1, gw2[...], preferred_element_type=f32) + gb2[...]).astype(bf)
    op = (jnp.dot(o1, ow2[...], preferred_element_type=f32) + ob2[...]).astype(bf)
    msg_out[...] = (_silu_t(op) * _sigmoid_t(gp)).astype(f32) * ew[...]


def _edge_mlp(gathered, edge_feat, edge_weight, weights, n_edges, blk_off):
    n_blocks = n_edges // BLK
    dst_block_off = n_edges // BLK

    def full(w):
        return pl.BlockSpec(w.shape, lambda i: tuple(0 for _ in w.shape))

    w_specs = [full(w) for w in weights]
    return pl.pallas_call(
        _mlp_body,
        grid=(n_blocks,),
        in_specs=[
            pl.BlockSpec((BLK, TW), lambda i: (i, 0)),
            pl.BlockSpec((BLK, TW), lambda i: (i + dst_block_off, 0)),
            pl.BlockSpec((BLK, ED), lambda i: (i + blk_off, 0)),
            pl.BlockSpec((BLK, D), lambda i: (i + blk_off, 0)),
            *w_specs,
        ],
        out_specs=pl.BlockSpec((BLK, D), lambda i: (i, 0)),
        out_shape=jax.ShapeDtypeStruct((n_edges, D), jnp.float32),
    )(gathered, gathered, edge_feat, edge_weight, *weights)


NBLK = 2000


def _final_body(n_parts, *refs):
    node_feat = refs[0]
    parts = refs[1:1 + n_parts]
    lin_w, lin_b, out = refs[1 + n_parts:]
    agg = parts[0][0]
    for p in parts:
        for j in range(NC):
            if p is parts[0] and j == 0:
                continue
            agg = agg + p[j]
    out[...] = node_feat[...] + jnp.dot(
        agg, lin_w[...], preferred_element_type=jnp.float32) + lin_b[...]


def _final_linear(node_feat, partials_list, lin_w, lin_b):
    n_blocks = N_NODES // NBLK
    part_specs = [pl.BlockSpec((NC, NBLK, D), lambda i: (0, i, 0))
                  for _ in partials_list]
    return pl.pallas_call(
        functools.partial(_final_body, len(partials_list)),
        grid=(n_blocks,),
        in_specs=[
            pl.BlockSpec((NBLK, D), lambda i: (i, 0)),
            *part_specs,
            pl.BlockSpec((D, D), lambda i: (0, 0)),
            pl.BlockSpec((1, D), lambda i: (0, 0)),
        ],
        out_specs=pl.BlockSpec((NBLK, D), lambda i: (i, 0)),
        out_shape=jax.ShapeDtypeStruct((N_NODES, D), jnp.float32),
    )(node_feat, *partials_list, lin_w, lin_b)


def kernel(node_feat, edge_feat, edge_weight, edge_index,
           g_W0, g_b0, g_W1, g_b1, g_W2, g_b2,
           o_W0, o_b0, o_W1, o_b1, o_W2, o_b2,
           lin_W, lin_b):
    bf = jnp.bfloat16
    node_packed = lax.bitcast_convert_type(
        node_feat.astype(bf).reshape(N_NODES, D // 2, 2), jnp.int32)

    w0 = jnp.concatenate([g_W0, o_W0], axis=1)
    w0s, w0d, w0e = w0[:D], w0[D:2 * D], w0[2 * D:]
    w0sd = jnp.concatenate(
        [w0s[0::2], w0s[1::2], w0d[0::2], w0d[1::2]], axis=0)
    b0 = jnp.concatenate([g_b0, o_b0]).reshape(1, 2 * H)
    weights = [w0sd.astype(bf), w0e.astype(bf), b0,
               g_W1.astype(bf), g_b1.reshape(1, H), o_W1.astype(bf), o_b1.reshape(1, H),
               g_W2.astype(bf), g_b2.reshape(1, D), o_W2.astype(bf), o_b2.reshape(1, D)]

    idx32 = edge_index.astype(jnp.int32)
    ef = edge_feat.astype(bf)
    zeros_stripe = jnp.zeros((SCH, D), jnp.float32)
    grp = NW * GCH * 2
    part = N_EDGES // SPLIT
    partials_list = []
    for h in range(SPLIT):
        lo = h * part
        idx_h = jnp.concatenate([idx32[0, lo:lo + part], idx32[1, lo:lo + part]])
        b_pad = grp * ((2 * part + grp - 1) // grp)
        idx_pad = jnp.concatenate(
            [idx_h, jnp.zeros((b_pad - 2 * part,), jnp.int32)])
        gathered = _sc_gather(node_packed, idx_pad)
        msg = _edge_mlp(gathered, ef, edge_weight, weights,
                        part, lo // BLK)
        partials_list.append(
            _sc_segment_sum(msg, idx32[1, lo:lo + part], zeros_stripe))

    return _final_linear(node_feat, partials_list, lin_W, lin_b.reshape(1, D))


# speedup vs baseline: 1.1961x; 1.1961x over previous
import functools

import jax
import jax.numpy as jnp
from jax import lax
from jax.experimental import pallas as pl
from jax.experimental.pallas import tpu as pltpu
from jax.experimental.pallas import tpu_sc as plsc

N_NODES = 10000
N_EDGES = 320000
D = 128
ED = 16
H = 256

NC = 2
NS = 16
NW = NC * NS
SPLIT = 1

GCH = 128


def _sc_gather_body(table_hbm, idx_hbm, out_hbm,
                    idx_all, rows_a, rows_b,
                    sem_ga, sem_gb, sem_wa, sem_wb):
    c = lax.axis_index("c")
    s = lax.axis_index("s")
    wid = s * NC + c
    n_total = idx_hbm.shape[0]
    per_w = n_total // NW
    base = wid * per_w
    n_pairs = per_w // GCH // 2

    def chunk(j):
        return pl.ds(pl.multiple_of(base + j * GCH, GCH), GCH)

    def idx_at(j):
        return idx_all.at[pl.ds(pl.multiple_of(j * GCH, GCH), GCH)]

    pltpu.sync_copy(idx_hbm.at[pl.ds(pl.multiple_of(base, GCH), per_w)],
                    idx_all)
    pltpu.async_copy(table_hbm.at[idx_at(0)], rows_a, sem_ga)

    @pl.loop(0, n_pairs)
    def _(jj):
        j = jj * 2

        @pl.when(jj > 0)
        def _():
            pltpu.make_async_copy(rows_b, out_hbm.at[chunk(j - 1)], sem_wb).wait()

        pltpu.async_copy(table_hbm.at[idx_at(j + 1)], rows_b, sem_gb)
        pltpu.make_async_copy(table_hbm.at[idx_at(j)], rows_a, sem_ga).wait()
        pltpu.async_copy(rows_a, out_hbm.at[chunk(j)], sem_wa)

        @pl.when(jj < n_pairs - 1)
        def _():
            pltpu.make_async_copy(rows_a, out_hbm.at[chunk(j)], sem_wa).wait()
            pltpu.async_copy(table_hbm.at[idx_at(j + 2)], rows_a, sem_ga)

        pltpu.make_async_copy(table_hbm.at[idx_at(j + 1)], rows_b, sem_gb).wait()
        pltpu.async_copy(rows_b, out_hbm.at[chunk(j + 1)], sem_wb)

    pltpu.make_async_copy(rows_a, out_hbm.at[chunk(0)], sem_wa).wait()
    pltpu.make_async_copy(rows_b, out_hbm.at[chunk(1)], sem_wb).wait()


TW = 64


def _sc_gather(table, idx):
    b = idx.shape[0]
    per_w = b // NW
    mesh = plsc.VectorSubcoreMesh(core_axis_name="c", subcore_axis_name="s")
    return pl.kernel(
        _sc_gather_body,
        out_type=jax.ShapeDtypeStruct((b, TW), jnp.int32),
        mesh=mesh,
        compiler_params=pltpu.CompilerParams(use_tc_tiling_on_sc=False),
        scratch_types=[
            pltpu.VMEM((per_w,), jnp.int32),
            pltpu.VMEM((GCH, TW), jnp.int32),
            pltpu.VMEM((GCH, TW), jnp.int32),
            pltpu.SemaphoreType.DMA,
            pltpu.SemaphoreType.DMA,
            pltpu.SemaphoreType.DMA,
            pltpu.SemaphoreType.DMA,
        ],
    )(table, idx)


SCH = 80
N_PAD = 10240
STRIPE = N_PAD // NS


def _sc_scatter_body(n_edges, msg_hbm, dst_hbm, zeros_hbm, out_hbm,
                     idx_all, rows_a, rows_b, acc_sh, sem_ra, sem_rb):
    c = lax.axis_index("c")
    s = lax.axis_index("s")
    per_core = n_edges // NC
    per_tile = per_core // NS
    base = c * per_core + s * per_tile
    n_chunks = per_tile // SCH
    n_pairs = n_chunks // 2

    stripe = pl.ds(pl.multiple_of(s * STRIPE, 8), STRIPE)

    def chunk(j):
        return pl.ds(pl.multiple_of(base + j * SCH, 8), SCH)

    def idx_at(j):
        return idx_all.at[pl.ds(pl.multiple_of(j * SCH, 8), SCH)]

    pltpu.sync_copy(zeros_hbm, acc_sh.at[stripe])
    pltpu.sync_copy(dst_hbm.at[pl.ds(pl.multiple_of(base, 8), per_tile)],
                    idx_all)
    plsc.subcore_barrier()

    pltpu.async_copy(msg_hbm.at[chunk(0)], rows_a, sem_ra)

    @pl.loop(0, n_pairs)
    def _(jj):
        j = jj * 2
        pltpu.async_copy(msg_hbm.at[chunk(j + 1)], rows_b, sem_rb)
        pltpu.make_async_copy(msg_hbm.at[chunk(j)], rows_a, sem_ra).wait()
        pltpu.sync_copy(rows_a, acc_sh.at[idx_at(j)], add=True)

        @pl.when(j + 2 < n_chunks)
        def _():
            pltpu.async_copy(msg_hbm.at[chunk(j + 2)], rows_a, sem_ra)

        pltpu.make_async_copy(msg_hbm.at[chunk(j + 1)], rows_b, sem_rb).wait()
        pltpu.sync_copy(rows_b, acc_sh.at[idx_at(j + 1)], add=True)

    if n_chunks % 2 == 1:
        j = n_chunks - 1
        pltpu.make_async_copy(msg_hbm.at[chunk(j)], rows_a, sem_ra).wait()
        pltpu.sync_copy(rows_a, acc_sh.at[idx_at(j)], add=True)

    plsc.subcore_barrier()

    pltpu.sync_copy(acc_sh.at[stripe], out_hbm.at[c, stripe])


def _sc_segment_sum(msg, dst_idx, zeros_stripe):
    per_tile = msg.shape[0] // NW
    mesh = plsc.VectorSubcoreMesh(core_axis_name="c", subcore_axis_name="s")
    return pl.kernel(
        functools.partial(_sc_scatter_body, msg.shape[0]),
        out_type=jax.ShapeDtypeStruct((NC, N_PAD, D), jnp.float32),
        mesh=mesh,
        scratch_types=[
            pltpu.VMEM((per_tile,), jnp.int32),
            pltpu.VMEM((SCH, D), jnp.float32),
            pltpu.VMEM((SCH, D), jnp.float32),
            pltpu.VMEM_SHARED((N_PAD, D), jnp.float32),
            pltpu.SemaphoreType.DMA,
            pltpu.SemaphoreType.DMA,
        ],
    )(msg, dst_idx, zeros_stripe)


BLK = 1280


def _silu(x):
    return x * jax.nn.sigmoid(x)


def _unpack_pairs(x_i32):
    f32 = jnp.float32
    bf = jnp.bfloat16
    even = lax.bitcast_convert_type(jnp.left_shift(x_i32, 16), f32).astype(bf)
    odd = lax.bitcast_convert_type(
        jnp.bitwise_and(x_i32, jnp.int32(-65536)), f32).astype(bf)
    return jnp.concatenate([even, odd], axis=1)


def _sigmoid_t(x):
    return 0.5 * jnp.tanh(x * 0.5) + 0.5


def _silu_t(x):
    return x * _sigmoid_t(x)


def _mlp_body(src, dst, ef, ew,
              w0sd, w0e, b0, gw1, gb1, ow1, ob1, gw2, gb2, ow2, ob2,
              msg_out):
    f32 = jnp.float32
    bf = jnp.bfloat16
    x = jnp.concatenate(
        [_unpack_pairs(src[...]), _unpack_pairs(dst[...])], axis=1)
    pre0 = (jnp.dot(x, w0sd[...], preferred_element_type=f32)
            + jnp.dot(ef[...], w0e[...], preferred_element_type=f32)
            + b0[...])
    a1 = _silu_t(pre0.astype(bf))
    g1 = _silu_t((jnp.dot(a1[:, :H], gw1[...], preferred_element_type=f32) + gb1[...]).astype(bf))
    o1 = _silu_t((jnp.dot(a1[:, H:], ow1[...], preferred_element_type=f32) + ob1[...]).astype(bf))
    gp = (jnp.dot(g1, gw2[...], preferred_element_type=f32) + gb2[...]).astype(bf)
    op = (jnp.dot(o1, ow2[...], preferred_element_type=f32) + ob2[...]).astype(bf)
    msg_out[...] = (_silu_t(op) * _sigmoid_t(gp)).astype(f32) * ew[...]


def _edge_mlp(gathered, edge_feat, edge_weight, weights, n_edges, blk_off):
    n_blocks = n_edges // BLK
    dst_block_off = n_edges // BLK

    def full(w):
        return pl.BlockSpec(w.shape, lambda i: tuple(0 for _ in w.shape))

    w_specs = [full(w) for w in weights]
    return pl.pallas_call(
        _mlp_body,
        grid=(n_blocks,),
        in_specs=[
            pl.BlockSpec((BLK, TW), lambda i: (i, 0)),
            pl.BlockSpec((BLK, TW), lambda i: (i + dst_block_off, 0)),
            pl.BlockSpec((BLK, ED), lambda i: (i + blk_off, 0)),
            pl.BlockSpec((BLK, D), lambda i: (i + blk_off, 0)),
            *w_specs,
        ],
        out_specs=pl.BlockSpec((BLK, D), lambda i: (i, 0)),
        out_shape=jax.ShapeDtypeStruct((n_edges, D), jnp.float32),
    )(gathered, gathered, edge_feat, edge_weight, *weights)


NBLK = 2000


def _final_body(n_parts, *refs):
    node_feat = refs[0]
    parts = refs[1:1 + n_parts]
    lin_w, lin_b, out = refs[1 + n_parts:]
    agg = parts[0][0]
    for p in parts:
        for j in range(NC):
            if p is parts[0] and j == 0:
                continue
            agg = agg + p[j]
    out[...] = node_feat[...] + jnp.dot(
        agg, lin_w[...], preferred_element_type=jnp.float32) + lin_b[...]


def _final_linear(node_feat, partials_list, lin_w, lin_b):
    n_blocks = N_NODES // NBLK
    part_specs = [pl.BlockSpec((NC, NBLK, D), lambda i: (0, i, 0))
                  for _ in partials_list]
    return pl.pallas_call(
        functools.partial(_final_body, len(partials_list)),
        grid=(n_blocks,),
        in_specs=[
            pl.BlockSpec((NBLK, D), lambda i: (i, 0)),
            *part_specs,
            pl.BlockSpec((D, D), lambda i: (0, 0)),
            pl.BlockSpec((1, D), lambda i: (0, 0)),
        ],
        out_specs=pl.BlockSpec((NBLK, D), lambda i: (i, 0)),
        out_shape=jax.ShapeDtypeStruct((N_NODES, D), jnp.float32),
    )(node_feat, *partials_list, lin_w, lin_b)


def kernel(node_feat, edge_feat, edge_weight, edge_index,
           g_W0, g_b0, g_W1, g_b1, g_W2, g_b2,
           o_W0, o_b0, o_W1, o_b1, o_W2, o_b2,
           lin_W, lin_b):
    bf = jnp.bfloat16
    node_packed = lax.bitcast_convert_type(
        node_feat.astype(bf).reshape(N_NODES, D // 2, 2), jnp.int32)

    w0 = jnp.concatenate([g_W0, o_W0], axis=1)
    w0s, w0d, w0e = w0[:D], w0[D:2 * D], w0[2 * D:]
    w0sd = jnp.concatenate(
        [w0s[0::2], w0s[1::2], w0d[0::2], w0d[1::2]], axis=0)
    b0 = jnp.concatenate([g_b0, o_b0]).reshape(1, 2 * H)
    weights = [w0sd.astype(bf), w0e.astype(bf), b0,
               g_W1.astype(bf), g_b1.reshape(1, H), o_W1.astype(bf), o_b1.reshape(1, H),
               g_W2.astype(bf), g_b2.reshape(1, D), o_W2.astype(bf), o_b2.reshape(1, D)]

    idx32 = edge_index.astype(jnp.int32)
    ef = edge_feat.astype(bf)
    zeros_stripe = jnp.zeros((STRIPE, D), jnp.float32)
    grp = NW * GCH * 2
    part = N_EDGES // SPLIT
    partials_list = []
    for h in range(SPLIT):
        lo = h * part
        idx_h = jnp.concatenate([idx32[0, lo:lo + part], idx32[1, lo:lo + part]])
        b_pad = grp * ((2 * part + grp - 1) // grp)
        idx_pad = jnp.concatenate(
            [idx_h, jnp.zeros((b_pad - 2 * part,), jnp.int32)])
        gathered = _sc_gather(node_packed, idx_pad)
        msg = _edge_mlp(gathered, ef, edge_weight, weights,
                        part, lo // BLK)
        partials_list.append(
            _sc_segment_sum(msg, idx32[1, lo:lo + part], zeros_stripe))

    return _final_linear(node_feat, partials_list, lin_W, lin_b.reshape(1, D))


# speedup vs baseline: 1.2371x; 1.0343x over previous
import functools

import jax
import jax.numpy as jnp
from jax import lax
from jax.experimental import pallas as pl
from jax.experimental.pallas import tpu as pltpu
from jax.experimental.pallas import tpu_sc as plsc

N_NODES = 10000
N_EDGES = 320000
D = 128
ED = 16
H = 256

NC = 2
NS = 16
NW = NC * NS
SPLIT = 1

GCH = 128


def _sc_gather_body(table_hbm, idx_hbm, out_hbm,
                    idx_all, rows_a, rows_b,
                    sem_ga, sem_gb, sem_wa, sem_wb):
    c = lax.axis_index("c")
    s = lax.axis_index("s")
    wid = s * NC + c
    n_total = idx_hbm.shape[0]
    per_w = n_total // NW
    base = wid * per_w
    n_pairs = per_w // GCH // 2

    def chunk(j):
        return pl.ds(pl.multiple_of(base + j * GCH, GCH), GCH)

    def idx_at(j):
        return idx_all.at[pl.ds(pl.multiple_of(j * GCH, GCH), GCH)]

    pltpu.sync_copy(idx_hbm.at[pl.ds(pl.multiple_of(base, GCH), per_w)],
                    idx_all)
    pltpu.async_copy(table_hbm.at[idx_at(0)], rows_a, sem_ga)

    @pl.loop(0, n_pairs)
    def _(jj):
        j = jj * 2

        @pl.when(jj > 0)
        def _():
            pltpu.make_async_copy(rows_b, out_hbm.at[chunk(j - 1)], sem_wb).wait()

        pltpu.async_copy(table_hbm.at[idx_at(j + 1)], rows_b, sem_gb)
        pltpu.make_async_copy(table_hbm.at[idx_at(j)], rows_a, sem_ga).wait()
        pltpu.async_copy(rows_a, out_hbm.at[chunk(j)], sem_wa)

        @pl.when(jj < n_pairs - 1)
        def _():
            pltpu.make_async_copy(rows_a, out_hbm.at[chunk(j)], sem_wa).wait()
            pltpu.async_copy(table_hbm.at[idx_at(j + 2)], rows_a, sem_ga)

        pltpu.make_async_copy(table_hbm.at[idx_at(j + 1)], rows_b, sem_gb).wait()
        pltpu.async_copy(rows_b, out_hbm.at[chunk(j + 1)], sem_wb)

    pltpu.make_async_copy(rows_a, out_hbm.at[chunk(0)], sem_wa).wait()
    pltpu.make_async_copy(rows_b, out_hbm.at[chunk(1)], sem_wb).wait()


TW = 64


def _sc_gather(table, idx):
    b = idx.shape[0]
    per_w = b // NW
    mesh = plsc.VectorSubcoreMesh(core_axis_name="c", subcore_axis_name="s")
    return pl.kernel(
        _sc_gather_body,
        out_type=jax.ShapeDtypeStruct((b, TW), jnp.int32),
        mesh=mesh,
        compiler_params=pltpu.CompilerParams(use_tc_tiling_on_sc=False),
        scratch_types=[
            pltpu.VMEM((per_w,), jnp.int32),
            pltpu.VMEM((GCH, TW), jnp.int32),
            pltpu.VMEM((GCH, TW), jnp.int32),
            pltpu.SemaphoreType.DMA,
            pltpu.SemaphoreType.DMA,
            pltpu.SemaphoreType.DMA,
            pltpu.SemaphoreType.DMA,
        ],
    )(table, idx)


SCH = 80
N_PAD = 10240
STRIPE = N_PAD // NS


def _sc_scatter_body(n_edges, msg_hbm, dst_hbm, zeros_hbm, out_hbm,
                     idx_all, rows_a, rows_b, acc_sh, sem_ra, sem_rb):
    c = lax.axis_index("c")
    s = lax.axis_index("s")
    per_core = n_edges // NC
    per_tile = per_core // NS
    base = c * per_core + s * per_tile
    n_chunks = per_tile // SCH
    n_pairs = n_chunks // 2

    stripe = pl.ds(pl.multiple_of(s * STRIPE, 8), STRIPE)

    def chunk(j):
        return pl.ds(pl.multiple_of(base + j * SCH, 8), SCH)

    def idx_at(j):
        return idx_all.at[pl.ds(pl.multiple_of(j * SCH, 8), SCH)]

    pltpu.sync_copy(zeros_hbm, acc_sh.at[stripe])
    pltpu.sync_copy(dst_hbm.at[pl.ds(pl.multiple_of(base, 8), per_tile)],
                    idx_all)
    plsc.subcore_barrier()

    pltpu.async_copy(msg_hbm.at[chunk(0)], rows_a, sem_ra)

    @pl.loop(0, n_pairs)
    def _(jj):
        j = jj * 2
        pltpu.async_copy(msg_hbm.at[chunk(j + 1)], rows_b, sem_rb)
        pltpu.make_async_copy(msg_hbm.at[chunk(j)], rows_a, sem_ra).wait()
        pltpu.sync_copy(rows_a, acc_sh.at[idx_at(j)], add=True)

        @pl.when(j + 2 < n_chunks)
        def _():
            pltpu.async_copy(msg_hbm.at[chunk(j + 2)], rows_a, sem_ra)

        pltpu.make_async_copy(msg_hbm.at[chunk(j + 1)], rows_b, sem_rb).wait()
        pltpu.sync_copy(rows_b, acc_sh.at[idx_at(j + 1)], add=True)

    if n_chunks % 2 == 1:
        j = n_chunks - 1
        pltpu.make_async_copy(msg_hbm.at[chunk(j)], rows_a, sem_ra).wait()
        pltpu.sync_copy(rows_a, acc_sh.at[idx_at(j)], add=True)

    plsc.subcore_barrier()

    pltpu.sync_copy(acc_sh.at[stripe], out_hbm.at[c, stripe])


def _sc_segment_sum(msg, dst_idx, zeros_stripe):
    per_tile = msg.shape[0] // NW
    mesh = plsc.VectorSubcoreMesh(core_axis_name="c", subcore_axis_name="s")
    return pl.kernel(
        functools.partial(_sc_scatter_body, msg.shape[0]),
        out_type=jax.ShapeDtypeStruct((NC, N_PAD, D), jnp.float32),
        mesh=mesh,
        scratch_types=[
            pltpu.VMEM((per_tile,), jnp.int32),
            pltpu.VMEM((SCH, D), jnp.float32),
            pltpu.VMEM((SCH, D), jnp.float32),
            pltpu.VMEM_SHARED((N_PAD, D), jnp.float32),
            pltpu.SemaphoreType.DMA,
            pltpu.SemaphoreType.DMA,
        ],
    )(msg, dst_idx, zeros_stripe)


BLK = 2560


def _unpack_pairs(x_i32):
    f32 = jnp.float32
    bf = jnp.bfloat16
    even = lax.bitcast_convert_type(jnp.left_shift(x_i32, 16), f32).astype(bf)
    odd = lax.bitcast_convert_type(
        jnp.bitwise_and(x_i32, jnp.int32(-65536)), f32).astype(bf)
    return jnp.concatenate([even, odd], axis=1)


def _half_silu(xh):
    return xh * (jnp.tanh(xh) + 1)


def _mlp_body(src, dst, ef, ew,
              w0sd, w0e, b0, gw1, gb1, ow1, ob1, gw2, gb2, ow2, ob2,
              msg_out):
    f32 = jnp.float32
    bf = jnp.bfloat16
    x = jnp.concatenate(
        [_unpack_pairs(src[...]), _unpack_pairs(dst[...])], axis=1)
    pre0 = (jnp.dot(x, w0sd[...], preferred_element_type=f32)
            + jnp.dot(ef[...], w0e[...], preferred_element_type=f32)
            + b0[...])
    a1 = _half_silu(pre0.astype(bf))
    g1 = _half_silu((jnp.dot(a1[:, :H], gw1[...], preferred_element_type=f32) + gb1[...]).astype(bf))
    o1 = _half_silu((jnp.dot(a1[:, H:], ow1[...], preferred_element_type=f32) + ob1[...]).astype(bf))
    gh = (jnp.dot(g1, gw2[...], preferred_element_type=f32) + gb2[...]).astype(bf)
    oh = (jnp.dot(o1, ow2[...], preferred_element_type=f32) + ob2[...]).astype(bf)
    prod = (oh * (jnp.tanh(oh) + 1)) * (jnp.tanh(gh) + 1)
    msg_out[...] = prod.astype(f32) * ew[...]


def _edge_mlp(gathered, edge_feat, edge_weight, weights, n_edges, blk_off):
    n_blocks = n_edges // BLK
    dst_block_off = n_edges // BLK

    def full(w):
        return pl.BlockSpec(w.shape, lambda i: tuple(0 for _ in w.shape))

    w_specs = [full(w) for w in weights]
    return pl.pallas_call(
        _mlp_body,
        grid=(n_blocks,),
        in_specs=[
            pl.BlockSpec((BLK, TW), lambda i: (i, 0)),
            pl.BlockSpec((BLK, TW), lambda i: (i + dst_block_off, 0)),
            pl.BlockSpec((BLK, ED), lambda i: (i + blk_off, 0)),
            pl.BlockSpec((BLK, D), lambda i: (i + blk_off, 0)),
            *w_specs,
        ],
        out_specs=pl.BlockSpec((BLK, D), lambda i: (i, 0)),
        out_shape=jax.ShapeDtypeStruct((n_edges, D), jnp.float32),
    )(gathered, gathered, edge_feat, edge_weight, *weights)


NBLK = 2000


def _final_body(n_parts, *refs):
    node_feat = refs[0]
    parts = refs[1:1 + n_parts]
    lin_w, lin_b, out = refs[1 + n_parts:]
    agg = parts[0][0]
    for p in parts:
        for j in range(NC):
            if p is parts[0] and j == 0:
                continue
            agg = agg + p[j]
    out[...] = node_feat[...] + jnp.dot(
        agg, lin_w[...], preferred_element_type=jnp.float32) + lin_b[...]


def _final_linear(node_feat, partials_list, lin_w, lin_b):
    n_blocks = N_NODES // NBLK
    part_specs = [pl.BlockSpec((NC, NBLK, D), lambda i: (0, i, 0))
                  for _ in partials_list]
    return pl.pallas_call(
        functools.partial(_final_body, len(partials_list)),
        grid=(n_blocks,),
        in_specs=[
            pl.BlockSpec((NBLK, D), lambda i: (i, 0)),
            *part_specs,
            pl.BlockSpec((D, D), lambda i: (0, 0)),
            pl.BlockSpec((1, D), lambda i: (0, 0)),
        ],
        out_specs=pl.BlockSpec((NBLK, D), lambda i: (i, 0)),
        out_shape=jax.ShapeDtypeStruct((N_NODES, D), jnp.float32),
    )(node_feat, *partials_list, lin_w, lin_b)


def kernel(node_feat, edge_feat, edge_weight, edge_index,
           g_W0, g_b0, g_W1, g_b1, g_W2, g_b2,
           o_W0, o_b0, o_W1, o_b1, o_W2, o_b2,
           lin_W, lin_b):
    bf = jnp.bfloat16
    node_packed = lax.bitcast_convert_type(
        node_feat.astype(bf).reshape(N_NODES, D // 2, 2), jnp.int32)

    w0 = jnp.concatenate([g_W0, o_W0], axis=1)
    w0s, w0d, w0e = w0[:D], w0[D:2 * D], w0[2 * D:]
    w0sd = jnp.concatenate(
        [w0s[0::2], w0s[1::2], w0d[0::2], w0d[1::2]], axis=0)
    b0 = jnp.concatenate([g_b0, o_b0]).reshape(1, 2 * H)
    weights = [w * 0.5 for w in (w0sd, w0e)] + [0.5 * b0]
    weights += [0.5 * g_W1, 0.5 * g_b1.reshape(1, H),
                0.5 * o_W1, 0.5 * o_b1.reshape(1, H),
                0.5 * g_W2, 0.5 * g_b2.reshape(1, D),
                0.5 * o_W2, 0.5 * o_b2.reshape(1, D)]
    weights = [w.astype(bf) if w.shape[0] != 1 else w for w in weights]

    idx32 = edge_index.astype(jnp.int32)
    ef = edge_feat.astype(bf)
    zeros_stripe = jnp.zeros((STRIPE, D), jnp.float32)
    grp = NW * GCH * 2
    part = N_EDGES // SPLIT
    partials_list = []
    for h in range(SPLIT):
        lo = h * part
        idx_h = jnp.concatenate([idx32[0, lo:lo + part], idx32[1, lo:lo + part]])
        b_pad = grp * ((2 * part + grp - 1) // grp)
        idx_pad = jnp.concatenate(
            [idx_h, jnp.zeros((b_pad - 2 * part,), jnp.int32)])
        gathered = _sc_gather(node_packed, idx_pad)
        msg = _edge_mlp(gathered, ef, edge_weight, weights,
                        part, lo // BLK)
        partials_list.append(
            _sc_segment_sum(msg, idx32[1, lo:lo + part], zeros_stripe))

    return _final_linear(node_feat, partials_list, 0.5 * lin_W,
                         lin_b.reshape(1, D))


# speedup vs baseline: 1.3508x; 1.0919x over previous
import functools

import jax
import jax.numpy as jnp
from jax import lax
from jax.experimental import pallas as pl
from jax.experimental.pallas import tpu as pltpu
from jax.experimental.pallas import tpu_sc as plsc

N_NODES = 10000
N_EDGES = 320000
D = 128
ED = 16
H = 256

NC = 2
NS = 16
NW = NC * NS
SPLIT = 1

GCH = 128


def _sc_gather_body(table_hbm, idx_hbm, out_hbm,
                    idx_all, rows_a, rows_b,
                    sem_ga, sem_gb, sem_wa, sem_wb):
    c = lax.axis_index("c")
    s = lax.axis_index("s")
    wid = s * NC + c
    n_total = idx_hbm.shape[0]
    per_w = n_total // NW
    base = wid * per_w
    n_pairs = per_w // GCH // 2

    def chunk(j):
        return pl.ds(pl.multiple_of(base + j * GCH, GCH), GCH)

    def idx_at(j):
        return idx_all.at[pl.ds(pl.multiple_of(j * GCH, GCH), GCH)]

    pltpu.sync_copy(idx_hbm.at[pl.ds(pl.multiple_of(base, GCH), per_w)],
                    idx_all)
    pltpu.async_copy(table_hbm.at[idx_at(0)], rows_a, sem_ga)

    @pl.loop(0, n_pairs)
    def _(jj):
        j = jj * 2

        @pl.when(jj > 0)
        def _():
            pltpu.make_async_copy(rows_b, out_hbm.at[chunk(j - 1)], sem_wb).wait()

        pltpu.async_copy(table_hbm.at[idx_at(j + 1)], rows_b, sem_gb)
        pltpu.make_async_copy(table_hbm.at[idx_at(j)], rows_a, sem_ga).wait()
        pltpu.async_copy(rows_a, out_hbm.at[chunk(j)], sem_wa)

        @pl.when(jj < n_pairs - 1)
        def _():
            pltpu.make_async_copy(rows_a, out_hbm.at[chunk(j)], sem_wa).wait()
            pltpu.async_copy(table_hbm.at[idx_at(j + 2)], rows_a, sem_ga)

        pltpu.make_async_copy(table_hbm.at[idx_at(j + 1)], rows_b, sem_gb).wait()
        pltpu.async_copy(rows_b, out_hbm.at[chunk(j + 1)], sem_wb)

    pltpu.make_async_copy(rows_a, out_hbm.at[chunk(0)], sem_wa).wait()
    pltpu.make_async_copy(rows_b, out_hbm.at[chunk(1)], sem_wb).wait()


TW = 64


def _sc_gather(table, idx):
    b = idx.shape[0]
    per_w = b // NW
    mesh = plsc.VectorSubcoreMesh(core_axis_name="c", subcore_axis_name="s")
    return pl.kernel(
        _sc_gather_body,
        out_type=jax.ShapeDtypeStruct((b, TW), jnp.int32),
        mesh=mesh,
        compiler_params=pltpu.CompilerParams(use_tc_tiling_on_sc=False),
        scratch_types=[
            pltpu.VMEM((per_w,), jnp.int32),
            pltpu.VMEM((GCH, TW), jnp.int32),
            pltpu.VMEM((GCH, TW), jnp.int32),
            pltpu.SemaphoreType.DMA,
            pltpu.SemaphoreType.DMA,
            pltpu.SemaphoreType.DMA,
            pltpu.SemaphoreType.DMA,
        ],
    )(table, idx)


SCH = 80
N_PAD = 10240
STRIPE = N_PAD // NS


def _sc_scatter_body(n_edges, msg_hbm, dst_hbm, zeros_hbm, out_hbm,
                     idx_all, rows_a, rows_b, acc_sh, sem_ra, sem_rb):
    c = lax.axis_index("c")
    s = lax.axis_index("s")
    per_core = n_edges // NC
    per_tile = per_core // NS
    base = c * per_core + s * per_tile
    n_chunks = per_tile // SCH
    n_pairs = n_chunks // 2

    stripe = pl.ds(pl.multiple_of(s * STRIPE, 8), STRIPE)

    def chunk(j):
        return pl.ds(pl.multiple_of(base + j * SCH, 8), SCH)

    def idx_at(j):
        return idx_all.at[pl.ds(pl.multiple_of(j * SCH, 8), SCH)]

    pltpu.sync_copy(zeros_hbm, acc_sh.at[stripe])
    pltpu.sync_copy(dst_hbm.at[pl.ds(pl.multiple_of(base, 8), per_tile)],
                    idx_all)
    plsc.subcore_barrier()

    pltpu.async_copy(msg_hbm.at[chunk(0)], rows_a, sem_ra)

    @pl.loop(0, n_pairs)
    def _(jj):
        j = jj * 2
        pltpu.async_copy(msg_hbm.at[chunk(j + 1)], rows_b, sem_rb)
        pltpu.make_async_copy(msg_hbm.at[chunk(j)], rows_a, sem_ra).wait()
        pltpu.sync_copy(rows_a, acc_sh.at[idx_at(j)], add=True)

        @pl.when(j + 2 < n_chunks)
        def _():
            pltpu.async_copy(msg_hbm.at[chunk(j + 2)], rows_a, sem_ra)

        pltpu.make_async_copy(msg_hbm.at[chunk(j + 1)], rows_b, sem_rb).wait()
        pltpu.sync_copy(rows_b, acc_sh.at[idx_at(j + 1)], add=True)

    if n_chunks % 2 == 1:
        j = n_chunks - 1
        pltpu.make_async_copy(msg_hbm.at[chunk(j)], rows_a, sem_ra).wait()
        pltpu.sync_copy(rows_a, acc_sh.at[idx_at(j)], add=True)

    plsc.subcore_barrier()

    pltpu.sync_copy(acc_sh.at[stripe], out_hbm.at[c, stripe])


def _sc_segment_sum(msg, dst_idx, zeros_stripe):
    per_tile = msg.shape[0] // NW
    mesh = plsc.VectorSubcoreMesh(core_axis_name="c", subcore_axis_name="s")
    return pl.kernel(
        functools.partial(_sc_scatter_body, msg.shape[0]),
        out_type=jax.ShapeDtypeStruct((NC, N_PAD, D), jnp.float32),
        mesh=mesh,
        scratch_types=[
            pltpu.VMEM((per_tile,), jnp.int32),
            pltpu.VMEM((SCH, D), jnp.float32),
            pltpu.VMEM((SCH, D), jnp.float32),
            pltpu.VMEM_SHARED((N_PAD, D), jnp.float32),
            pltpu.SemaphoreType.DMA,
            pltpu.SemaphoreType.DMA,
        ],
    )(msg, dst_idx, zeros_stripe)


BLK = 2560


def _unpack_pairs(x_i32):
    f32 = jnp.float32
    bf = jnp.bfloat16
    even = lax.bitcast_convert_type(jnp.left_shift(x_i32, 16), f32).astype(bf)
    odd = lax.bitcast_convert_type(
        jnp.bitwise_and(x_i32, jnp.int32(-65536)), f32).astype(bf)
    return jnp.concatenate([even, odd], axis=1)


def _half_silu(xh):
    return xh * (jnp.tanh(xh) + 1)


def _mlp_body(sd, ef, ew,
              w0sd, w0e, b0, gw1, gb1, ow1, ob1, gw2, gb2, ow2, ob2,
              msg_out):
    f32 = jnp.float32
    bf = jnp.bfloat16
    x = _unpack_pairs(sd[...])
    pre0 = (jnp.dot(x, w0sd[...], preferred_element_type=f32)
            + jnp.dot(ef[...], w0e[...], preferred_element_type=f32)
            + b0[...])
    a1 = _half_silu(pre0.astype(bf))
    g1 = _half_silu((jnp.dot(a1[:, :H], gw1[...], preferred_element_type=f32) + gb1[...]).astype(bf))
    o1 = _half_silu((jnp.dot(a1[:, H:], ow1[...], preferred_element_type=f32) + ob1[...]).astype(bf))
    gh = (jnp.dot(g1, gw2[...], preferred_element_type=f32) + gb2[...]).astype(bf)
    oh = (jnp.dot(o1, ow2[...], preferred_element_type=f32) + ob2[...]).astype(bf)
    prod = (oh * (jnp.tanh(oh) + 1)) * (jnp.tanh(gh) + 1)
    msg_out[...] = prod.astype(f32) * ew[...]


def _edge_mlp(gathered, edge_feat, edge_weight, weights, n_edges, blk_off):
    n_blocks = n_edges // BLK

    def full(w):
        return pl.BlockSpec(w.shape, lambda i: tuple(0 for _ in w.shape))

    w_specs = [full(w) for w in weights]
    return pl.pallas_call(
        _mlp_body,
        grid=(n_blocks,),
        in_specs=[
            pl.BlockSpec((BLK, 2 * TW), lambda i: (i, 0)),
            pl.BlockSpec((BLK, ED), lambda i: (i + blk_off, 0)),
            pl.BlockSpec((BLK, D), lambda i: (i + blk_off, 0)),
            *w_specs,
        ],
        out_specs=pl.BlockSpec((BLK, D), lambda i: (i, 0)),
        out_shape=jax.ShapeDtypeStruct((n_edges, D), jnp.float32),
    )(gathered, edge_feat, edge_weight, *weights)


NBLK = 2000


def _final_body(n_parts, *refs):
    node_feat = refs[0]
    parts = refs[1:1 + n_parts]
    lin_w, lin_b, out = refs[1 + n_parts:]
    agg = parts[0][0]
    for p in parts:
        for j in range(NC):
            if p is parts[0] and j == 0:
                continue
            agg = agg + p[j]
    out[...] = node_feat[...] + jnp.dot(
        agg, lin_w[...], preferred_element_type=jnp.float32) + lin_b[...]


def _final_linear(node_feat, partials_list, lin_w, lin_b):
    n_blocks = N_NODES // NBLK
    part_specs = [pl.BlockSpec((NC, NBLK, D), lambda i: (0, i, 0))
                  for _ in partials_list]
    return pl.pallas_call(
        functools.partial(_final_body, len(partials_list)),
        grid=(n_blocks,),
        in_specs=[
            pl.BlockSpec((NBLK, D), lambda i: (i, 0)),
            *part_specs,
            pl.BlockSpec((D, D), lambda i: (0, 0)),
            pl.BlockSpec((1, D), lambda i: (0, 0)),
        ],
        out_specs=pl.BlockSpec((NBLK, D), lambda i: (i, 0)),
        out_shape=jax.ShapeDtypeStruct((N_NODES, D), jnp.float32),
    )(node_feat, *partials_list, lin_w, lin_b)


def kernel(node_feat, edge_feat, edge_weight, edge_index,
           g_W0, g_b0, g_W1, g_b1, g_W2, g_b2,
           o_W0, o_b0, o_W1, o_b1, o_W2, o_b2,
           lin_W, lin_b):
    bf = jnp.bfloat16
    node_packed = lax.bitcast_convert_type(
        node_feat.astype(bf).reshape(N_NODES, D // 2, 2), jnp.int32)

    w0 = jnp.concatenate([g_W0, o_W0], axis=1)
    w0s, w0d, w0e = w0[:D], w0[D:2 * D], w0[2 * D:]
    w0sd = jnp.concatenate(
        [w0s[0::2], w0d[0::2], w0s[1::2], w0d[1::2]], axis=0)
    b0 = jnp.concatenate([g_b0, o_b0]).reshape(1, 2 * H)
    weights = [w * 0.5 for w in (w0sd, w0e)] + [0.5 * b0]
    weights += [0.5 * g_W1, 0.5 * g_b1.reshape(1, H),
                0.5 * o_W1, 0.5 * o_b1.reshape(1, H),
                0.5 * g_W2, 0.5 * g_b2.reshape(1, D),
                0.5 * o_W2, 0.5 * o_b2.reshape(1, D)]
    weights = [w.astype(bf) if w.shape[0] != 1 else w for w in weights]

    idx32 = edge_index.astype(jnp.int32)
    ef = edge_feat.astype(bf)
    zeros_stripe = jnp.zeros((STRIPE, D), jnp.float32)
    grp = NW * GCH * 2
    part = N_EDGES // SPLIT
    partials_list = []
    for h in range(SPLIT):
        lo = h * part
        idx_h = idx32[:, lo:lo + part].T.reshape(-1)
        b_pad = grp * ((2 * part + grp - 1) // grp)
        idx_pad = jnp.concatenate(
            [idx_h, jnp.zeros((b_pad - 2 * part,), jnp.int32)])
        gathered = _sc_gather(node_packed, idx_pad).reshape(b_pad // 2, 2 * TW)
        msg = _edge_mlp(gathered, ef, edge_weight, weights,
                        part, lo // BLK)
        partials_list.append(
            _sc_segment_sum(msg, idx32[1, lo:lo + part], zeros_stripe))

    return _final_linear(node_feat, partials_list, 0.5 * lin_W,
                         lin_b.reshape(1, D))
